# baseline (speedup 1.0000x reference)
import functools

import jax
import jax.numpy as jnp
from jax import lax
from jax.experimental import pallas as pl
from jax.experimental.pallas import tpu as pltpu
from jax.experimental.pallas import tpu_sc as plsc

_NEG_RATIO = 3.0
_EPS = 1e-4

_R = 4096
_C = 1024

_SC_ROWS = 2048
_TC_ROWS = _R - _SC_ROWS
_BR = 512
_GRID = _TC_ROWS // _BR
_TC_BLK0 = _SC_ROWS // _BR

_NW = 32
_PER_W = _SC_ROWS * _C // _NW
_CHUNK = 8192
_NCH = _PER_W // _CHUNK

_LN_C = (1.4698117504763353e-06, 0.9998477529839026, -0.49737359923023405,
         0.31574842159182576, -0.19035583052804395, 0.08269215407154647,
         -0.017414274104031163)
_LN2 = 0.6931471805599453


def _loss_terms(pred, g, mask):
    p = jnp.where(g > 0.5, pred, 1.0 - pred)
    loss = -jnp.log(p)
    ml = loss * mask
    pos = g * mask
    return ml, pos


def _reduce_kernel(pa_ref, pb_ref, ga_ref, gb_ref, ma_ref, mb_ref, out_ref):
    i = pl.program_id(0)
    pos_sum = 0.0
    neg_sum = 0.0
    pcnt = 0.0
    mcnt = 0.0
    for p_ref, g_ref, m_ref in ((pa_ref, ga_ref, ma_ref),
                                (pb_ref, gb_ref, mb_ref)):
        pred = p_ref[...]
        g = g_ref[...]
        mask = m_ref[...]
        ml, pos = _loss_terms(pred, g, mask)
        pos_sum += jnp.sum(ml * g)
        neg_sum += jnp.sum(ml * (1.0 - g))
        pcnt += jnp.sum(pos)
        mcnt += jnp.sum(mask)

    @pl.when(i == 0)
    def _():
        out_ref[0] = 0.0
        out_ref[1] = 0.0
        out_ref[2] = 0.0
        out_ref[3] = 0.0

    out_ref[0] += pos_sum
    out_ref[1] += neg_sum
    out_ref[2] += pcnt
    out_ref[3] += mcnt - pcnt


def _sc_elem(p, g, m):
    x = jnp.where(g > 0.5, p, 1.0 - p)
    bits = lax.bitcast_convert_type(x, jnp.int32)
    e = lax.shift_right_logical(bits, 23) - 127
    ef = e.astype(jnp.float32)
    mant = lax.bitwise_or(lax.bitwise_and(bits, 0x7FFFFF), 0x3F800000)
    t = lax.bitcast_convert_type(mant, jnp.float32) - 1.0
    lnm = _LN_C[6]
    for c in (_LN_C[5], _LN_C[4], _LN_C[3], _LN_C[2], _LN_C[1], _LN_C[0]):
        lnm = lnm * t + c
    loss = -(ef * _LN2 + lnm)
    ml = loss * m
    mlg = ml * g
    return ml, mlg, g * m


_CROWS = _CHUNK // _C


def _sc_reduce_body(pred_h, gt_h, mask_h, out_h,
                    pb, gb, mb, acc_s, s0, s1, s2, s3, s4, s5):
    wid = lax.axis_index("s") * 2 + lax.axis_index("c")
    base = wid * (_PER_W // _C)
    sems = ((s0, s1, s2), (s3, s4, s5))

    def make(c):
        slot = c % 2
        off = base + c * _CROWS
        return [
            pltpu.make_async_copy(pred_h.at[pl.ds(off, _CROWS)], pb.at[slot],
                                  sems[slot][0]),
            pltpu.make_async_copy(gt_h.at[pl.ds(off, _CROWS)], gb.at[slot],
                                  sems[slot][1]),
            pltpu.make_async_copy(mask_h.at[pl.ds(off, _CROWS)], mb.at[slot],
                                  sems[slot][2]),
        ]

    zero = jnp.zeros((16,), jnp.float32)
    accs = (zero, zero, zero, zero)
    pending = {0: make(0)}
    for cp in pending[0]:
        cp.start()
    for c in range(_NCH):
        if c + 1 < _NCH:
            pending[c + 1] = make(c + 1)
            for cp in pending[c + 1]:
                cp.start()
        for cp in pending.pop(c):
            cp.wait()
        slot = c % 2

        def body(j, accs, slot=slot):
            ap, an, pc, mc = accs
            for r in range(_CROWS):
                for u in range(4):
                    sl = pl.ds((j * 4 + u) * 16, 16)
                    p = pb[slot, r, sl]
                    g = gb[slot, r, sl]
                    m = mb[slot, r, sl]
                    ml, mlg, pos = _sc_elem(p, g, m)
                    ap = ap + mlg
                    an = an + (ml - mlg)
                    pc = pc + pos
                    mc = mc + m
            return (ap, an, pc, mc)

        accs = lax.fori_loop(0, _C // 64, body, accs)

    ap, an, pc, mc = accs
    acc_s[0] = ap
    acc_s[1] = an
    acc_s[2] = pc
    acc_s[3] = mc - pc
    pltpu.sync_copy(acc_s, out_h.at[wid])


_sc_reduce = functools.partial(
    pl.kernel,
    out_type=jax.ShapeDtypeStruct((_NW, 4, 16), jnp.float32),
    mesh=plsc.VectorSubcoreMesh(core_axis_name="c", subcore_axis_name="s"),
    compiler_params=pltpu.CompilerParams(use_tc_tiling_on_sc=True),
    cost_estimate=pl.CostEstimate(
        flops=80_000_000, bytes_accessed=_SC_ROWS * _C * 12,
        transcendentals=0),
    scratch_types=[
        pltpu.VMEM((2, _CROWS, _C), jnp.float32),
        pltpu.VMEM((2, _CROWS, _C), jnp.float32),
        pltpu.VMEM((2, _CROWS, _C), jnp.float32),
        pltpu.VMEM((4, 16), jnp.float32),
        pltpu.SemaphoreType.DMA,
        pltpu.SemaphoreType.DMA,
        pltpu.SemaphoreType.DMA,
        pltpu.SemaphoreType.DMA,
        pltpu.SemaphoreType.DMA,
        pltpu.SemaphoreType.DMA,
    ],
)(_sc_reduce_body)


def _select_kernel(th_ref, pred_ref, gt_ref, mask_ref, out_ref):
    i = pl.program_id(0)
    t = th_ref[0]
    pred = pred_ref[...]
    g = gt_ref[...]
    mask = mask_ref[...]
    ml, _ = _loss_terms(pred, g, mask)
    nl = ml * (1.0 - g)
    bits = lax.bitcast_convert_type(nl, jnp.int32)
    ge = (bits >= t).astype(jnp.float32)
    gt_m = bits > t
    gtf = gt_m.astype(jnp.float32)

    @pl.when(i == 0)
    def _():
        out_ref[0] = 0.0
        out_ref[1] = 0.0
        out_ref[2] = 0.0

    out_ref[0] += jnp.sum(ge)
    out_ref[1] += jnp.sum(gtf)
    out_ref[2] += jnp.sum(jnp.where(gt_m, nl, 0.0))


def _in_specs(n, blk0=0):
    return [pl.BlockSpec((_BR, _C), lambda i: (i + blk0, 0)) for _ in range(n)]


def _run_reduce(p2, g2, m2):
    half_blocks = _GRID // 2
    specs = []
    for blk0 in (_TC_BLK0, _TC_BLK0 + half_blocks):
        specs.append(pl.BlockSpec((_BR, _C), lambda i, b=blk0: (i + b, 0)))
    in_specs = specs * 3
    return pl.pallas_call(
        _reduce_kernel,
        grid=(half_blocks,),
        in_specs=in_specs,
        out_specs=pl.BlockSpec(memory_space=pltpu.SMEM),
        out_shape=jax.ShapeDtypeStruct((4,), jnp.float32),
    )(p2, p2, g2, g2, m2, m2)


def _run_select(th, p2, g2, m2):
    return pl.pallas_call(
        _select_kernel,
        grid=(_R // _BR,),
        in_specs=[pl.BlockSpec(memory_space=pltpu.SMEM)] + _in_specs(3),
        out_specs=pl.BlockSpec(memory_space=pltpu.SMEM),
        out_shape=jax.ShapeDtypeStruct((3,), jnp.float32),
    )(th, p2, g2, m2)


def kernel(pred, gt, train_mask):
    p2 = pred.reshape(_R, _C)
    g2 = gt.reshape(_R, _C)
    m2 = train_mask.reshape(_R, _C)
    sc_out = _sc_reduce(p2, g2, m2)
    tc = _run_reduce(p2, g2, m2)
    sc = jnp.sum(sc_out, axis=(0, 2))
    pos_sum = tc[0] + sc[0]
    neg_sum = tc[1] + sc[1]
    pcnt = tc[2] + sc[2]
    ncnt = tc[3] + sc[3]
    k = jnp.minimum(ncnt, jnp.floor(pcnt * _NEG_RATIO))

    def common(_):
        return (pos_sum + neg_sum) / (pcnt + k + _EPS)

    def rare(_):
        def body(i, prefix):
            cand = prefix | (1 << (30 - i))
            s = _run_select(cand[None], p2, g2, m2)
            return jnp.where(s[0] >= k, cand, prefix)

        t = lax.fori_loop(0, 31, body, jnp.int32(0))
        s = _run_select(t[None], p2, g2, m2)
        tval = lax.bitcast_convert_type(t, jnp.float32)
        extra = k - s[1]
        neg_sel = s[2] + jnp.where(extra > 0, extra * tval, 0.0)
        return (pos_sum + neg_sel) / (pcnt + k + _EPS)

    return lax.cond(k >= ncnt, common, rare, None)


# speedup vs baseline: 1.4063x; 1.4063x over previous
import jax
import jax.numpy as jnp
from jax import lax
from jax.experimental import pallas as pl
from jax.experimental.pallas import tpu as pltpu

_NEG_RATIO = 3.0
_EPS = 1e-4

_R = 4096
_C = 1024
_GRID = 4
_BR = _R // _GRID


def _loss_terms(pred, g, mask):
    p = jnp.where(g > 0.5, pred, 1.0 - pred)
    loss = -jnp.log(p)
    ml = loss * mask
    pos = g * mask
    return ml, pos


def _reduce_kernel(pred_ref, gt_ref, mask_ref, out_ref):
    i = pl.program_id(0)
    pred = pred_ref[...]
    g = gt_ref[...]
    mask = mask_ref[...]
    ml, pos = _loss_terms(pred, g, mask)
    pos_sum = jnp.sum(ml * g)
    neg_sum = jnp.sum(ml * (1.0 - g))
    pcnt = jnp.sum(pos)
    mcnt = jnp.sum(mask)

    @pl.when(i == 0)
    def _():
        out_ref[1] = 0.0
        out_ref[2] = 0.0
        out_ref[3] = 0.0
        out_ref[4] = 0.0

    out_ref[1] += pos_sum
    out_ref[2] += neg_sum
    out_ref[3] += pcnt
    out_ref[4] += mcnt - pcnt

    @pl.when(i == _GRID - 1)
    def _():
        ps = out_ref[1]
        ns = out_ref[2]
        pc = out_ref[3]
        nc = out_ref[4]
        k = jnp.minimum(nc, jnp.floor(pc * _NEG_RATIO))
        out_ref[5] = k
        out_ref[0] = (ps + ns) / (pc + k + _EPS)


def _select_kernel(th_ref, pred_ref, gt_ref, mask_ref, out_ref):
    i = pl.program_id(0)
    t = th_ref[0]
    pred = pred_ref[...]
    g = gt_ref[...]
    mask = mask_ref[...]
    ml, _ = _loss_terms(pred, g, mask)
    nl = ml * (1.0 - g)
    bits = lax.bitcast_convert_type(nl, jnp.int32)
    ge = (bits >= t).astype(jnp.float32)
    gt_m = bits > t
    gtf = gt_m.astype(jnp.float32)

    @pl.when(i == 0)
    def _():
        out_ref[0] = 0.0
        out_ref[1] = 0.0
        out_ref[2] = 0.0

    out_ref[0] += jnp.sum(ge)
    out_ref[1] += jnp.sum(gtf)
    out_ref[2] += jnp.sum(jnp.where(gt_m, nl, 0.0))


def _in_specs(n):
    return [pl.BlockSpec((_BR, _C), lambda i: (i, 0)) for _ in range(n)]


def _run_reduce(p2, g2, m2):
    return pl.pallas_call(
        _reduce_kernel,
        grid=(_GRID,),
        in_specs=_in_specs(3),
        out_specs=pl.BlockSpec(memory_space=pltpu.SMEM),
        out_shape=jax.ShapeDtypeStruct((6,), jnp.float32),
    )(p2, g2, m2)


def _run_select(th, p2, g2, m2):
    return pl.pallas_call(
        _select_kernel,
        grid=(_GRID,),
        in_specs=[pl.BlockSpec(memory_space=pltpu.SMEM)] + _in_specs(3),
        out_specs=pl.BlockSpec(memory_space=pltpu.SMEM),
        out_shape=jax.ShapeDtypeStruct((3,), jnp.float32),
    )(th, p2, g2, m2)


def kernel(pred, gt, train_mask):
    p2 = pred.reshape(_R, _C)
    g2 = gt.reshape(_R, _C)
    m2 = train_mask.reshape(_R, _C)

    sums = _run_reduce(p2, g2, m2)
    res_common, pos_sum, pcnt, ncnt, k = sums[0], sums[1], sums[3], sums[4], sums[5]

    def common(_):
        return res_common

    def rare(_):
        def body(i, prefix):
            cand = prefix | (1 << (30 - i))
            s = _run_select(cand[None], p2, g2, m2)
            return jnp.where(s[0] >= k, cand, prefix)

        t = lax.fori_loop(0, 31, body, jnp.int32(0))
        s = _run_select(t[None], p2, g2, m2)
        tval = lax.bitcast_convert_type(t, jnp.float32)
        extra = k - s[1]
        neg_sel = s[2] + jnp.where(extra > 0, extra * tval, 0.0)
        return (pos_sum + neg_sel) / (pcnt + k + _EPS)

    return lax.cond(k >= ncnt, common, rare, None)
